# baseline (speedup 1.0000x reference)
import jax
import jax.numpy as jnp
from jax import lax
from jax.experimental import pallas as pl
from jax.experimental.pallas import tpu as pltpu
from jax.experimental.pallas import tpu_sc as plsc

BATCH = 16384
D = 64
L = 16
NC, NS = 2, 16
NW = NC * NS
BPW = BATCH // NW
CH = 128
NCH = BPW // CH
GROUPS = BPW // L

NROW = 1_000_000
BLK = 128
NBLK = (NROW + BLK - 1) // BLK
RPAD = NBLK * BLK
MAXB = (NBLK + NW - 1) // NW
LUTW = MAXB * BLK
IDXCH = 2048
RING = 8
TAILLO = (NBLK - 1) * BLK
NCNT = 2048


def _harvest(lut, buf, stg, gout_hbm, sem_out, loff, k, mcount):
    slots = lut[pl.ds(loff + k * L, L)]
    lane = lax.iota(jnp.int32, L)

    def cond(c):
        return jnp.any(c[0])

    def body(c):
        m, n = c
        ffs = plsc.all_reduce_ffs(m)
        bvec = jnp.where(lane == ffs, slots, -1)
        b_out = jnp.max(bvec)
        lvec = k * L + ffs
        slot = lax.rem(n, RING)
        for c4 in range(4):
            dvec = c4 * L + lane
            vals = plsc.load_gather(buf, [dvec, lvec])
            stg[pl.ds(slot * D + c4 * L, L)] = vals

        @pl.when(n >= RING)
        def _():
            pltpu.make_async_copy(gout_hbm.at[pl.ds(0, D)],
                                  stg.at[pl.ds(0, D)], sem_out).wait()

        pltpu.async_copy(stg.at[pl.ds(slot * D, D)],
                         gout_hbm.at[pl.ds(b_out * D, D)], sem_out)
        return m & (lane != ffs), n + 1

    m0 = slots >= 0
    _, mcount = lax.while_loop(cond, body, (m0, mcount))
    return mcount


def _stream_body(uidx_hbm, iidx_hbm, utab_hbm, itab_hbm,
                 win_u_hbm, win_i_hbm, gu_hbm, gi_hbm,
                 lut_u, lut_i, chunk_v, bufs_u, bufs_i, stg_u, stg_i,
                 bf_u, bf_i,
                 sem_in, sem_u, sem_i):
    wid = lax.axis_index("s") * NC + lax.axis_index("c")
    blo = (wid * NBLK) // NW
    bhi = ((wid + 1) * NBLK) // NW
    nblk = bhi - blo
    rlo = blo * BLK
    lane = lax.iota(jnp.int32, L)

    neg1 = jnp.full((L,), -1, jnp.int32)
    zero = jnp.zeros((L,), jnp.int32)

    def init(i, carry):
        lut_u[pl.ds(i * L, L)] = neg1
        lut_i[pl.ds(i * L, L)] = neg1
        return carry

    lax.fori_loop(0, LUTW // L, init, 0)

    def initf(i, carry):
        bf_u[pl.ds(i * L, L)] = zero
        bf_i[pl.ds(i * L, L)] = zero
        return carry

    lax.fori_loop(0, NCNT // L, initf, 0)

    rhi = rlo + LUTW

    one = jnp.full((L,), 1, jnp.int32)

    def scan_tab(idx_hbm, lut, bf):
        def chunk_loop(c, carry):
            pltpu.sync_copy(idx_hbm.at[pl.ds(c * IDXCH, IDXCH)], chunk_v)

            def vec_loop(i, carry2):
                v = chunk_v[pl.ds(i * L, L)]
                m = (v >= rlo) & (v < rhi)
                bvec = c * IDXCH + i * L + lane
                plsc.store_scatter(lut, [v - rlo], bvec, mask=m)
                plsc.addupdate_scatter(
                    bf, [lax.shift_right_logical(v - rlo, 4)], one, mask=m)
                return carry2

            return lax.fori_loop(0, IDXCH // L, vec_loop, carry)

        lax.fori_loop(0, BATCH // IDXCH, chunk_loop, 0)

    scan_tab(uidx_hbm, lut_u, bf_u)
    scan_tab(iidx_hbm, lut_i, bf_i)

    def flag_at(bf, j):
        base = j * 8
        b16 = (base // L) * L
        cv = bf[pl.ds(b16, L)]
        off = base - b16
        return jnp.max(jnp.where((lane >= off) & (lane < off + 8), cv, 0))

    base = 31232
    pltpu.sync_copy(lut_u.at[pl.ds(0, base)], win_u_hbm.at[pl.ds(rlo, base)])
    pltpu.sync_copy(lut_i.at[pl.ds(0, base)], win_i_hbm.at[pl.ds(rlo, base)])

    @pl.when(nblk == MAXB)
    def _():
        pltpu.sync_copy(lut_u.at[pl.ds(base, BLK)],
                        win_u_hbm.at[pl.ds(rlo + base, BLK)])
        pltpu.sync_copy(lut_i.at[pl.ds(base, BLK)],
                        win_i_hbm.at[pl.ds(rlo + base, BLK)])

    bhi_s = jnp.minimum(bhi, NBLK - 1)

    def fire1(tab_hbm, g, pbuf):
        pltpu.async_copy(tab_hbm.at[:, pl.ds(g * BLK, BLK)], pbuf, sem_in)

    @pl.when(flag_at(bf_u, 0) > 0)
    def _():
        fire1(utab_hbm, blo, bufs_u[0])

    @pl.when(flag_at(bf_i, 0) > 0)
    def _():
        fire1(itab_hbm, blo, bufs_i[0])

    def blk_loop(g, carry):
        mu, mi = carry
        lb = g - blo
        fu = flag_at(bf_u, lb)
        fi = flag_at(bf_i, lb)

        @pl.when(fu > 0)
        def _():
            pltpu.make_async_copy(utab_hbm.at[pl.ds(0, 64), pl.ds(0, BLK)],
                                  bufs_u[0], sem_in).wait()

        @pl.when(fi > 0)
        def _():
            pltpu.make_async_copy(utab_hbm.at[pl.ds(0, 64), pl.ds(0, BLK)],
                                  bufs_i[0], sem_in).wait()

        p = lax.rem(lb, 2)

        def do(pbuf_u, pbuf_i, mu, mi):
            @pl.when((g + 1 < bhi_s) & (flag_at(bf_u, lb + 1) > 0))
            def _():
                fire1(utab_hbm, g + 1, pbuf_u)

            @pl.when((g + 1 < bhi_s) & (flag_at(bf_i, lb + 1) > 0))
            def _():
                fire1(itab_hbm, g + 1, pbuf_i)
            return mu, mi

        def proc(pbuf_u, pbuf_i, mu, mi):
            loff = lb * BLK

            def hloop(lut, pbuf, stg, gout, sem, bf):
                base = lb * 8
                b16 = (base // L) * L
                cv = bf[pl.ds(b16, L)]
                off = base - b16
                gm = (cv > 0) & (lane >= off) & (lane < off + 8)

                def cond(c):
                    return jnp.any(c[0])

                def body(c):
                    gmm, mc = c
                    gffs = plsc.all_reduce_ffs(gmm)
                    k = jnp.max(gffs) - off
                    mc = _harvest(lut, pbuf, stg, gout, sem, loff, k, mc)
                    return gmm & (lane != gffs), mc

                def run(c):
                    _, mc = lax.while_loop(cond, body, (gm, c))
                    return mc

                return run

            mu = lax.cond(fu > 0,
                          hloop(lut_u, pbuf_u, stg_u, gu_hbm, sem_u, bf_u),
                          lambda c: c, mu)
            mi = lax.cond(fi > 0,
                          hloop(lut_i, pbuf_i, stg_i, gi_hbm, sem_i, bf_i),
                          lambda c: c, mi)
            return mu, mi

        def even(c):
            mu, mi = c
            mu, mi = do(bufs_u[1], bufs_i[1], mu, mi)
            return proc(bufs_u[0], bufs_i[0], mu, mi)

        def odd(c):
            mu, mi = c
            mu, mi = do(bufs_u[0], bufs_i[0], mu, mi)
            return proc(bufs_u[1], bufs_i[1], mu, mi)

        return lax.cond(p == 0, even, odd, (mu, mi))

    mu, mi = lax.fori_loop(blo, bhi_s, blk_loop,
                           (jnp.int32(0), jnp.int32(0)))

    def drain(sem, cnt, stg):
        def d(i, carry):
            pltpu.make_async_copy(gu_hbm.at[pl.ds(0, D)],
                                  stg.at[pl.ds(0, D)], sem).wait()
            return carry

        lax.fori_loop(0, jnp.minimum(cnt, RING), d, 0)

    drain(sem_u, mu, stg_u)
    drain(sem_i, mi, stg_i)


def _dot_body(uidx_hbm, iidx_hbm, win_u_hbm, win_i_hbm, gu_hbm, gi_hbm,
              tu_hbm, ti_hbm, ub_hbm, ib_hbm, gb_hbm, out_hbm,
              uidx_v, iidx_v, wu_v, wi_v, ru_v, ri_v, ubv, ibv, gbv, out_v,
              sem):
    wid = lax.axis_index("s") * NC + lax.axis_index("c")
    base = wid * BPW
    lane = lax.iota(jnp.int32, L)

    pltpu.sync_copy(uidx_hbm.at[pl.ds(base, BPW)], uidx_v)
    pltpu.sync_copy(iidx_hbm.at[pl.ds(base, BPW)], iidx_v)
    pltpu.sync_copy(gb_hbm, gbv)

    copies = []
    for j in range(NCH):
        s = pl.ds(j * CH, CH)
        copies.append(pltpu.async_copy(win_u_hbm.at[uidx_v.at[s]], wu_v.at[s], sem))
        copies.append(pltpu.async_copy(win_i_hbm.at[iidx_v.at[s]], wi_v.at[s], sem))
        copies.append(pltpu.async_copy(ub_hbm.at[uidx_v.at[s]], ubv.at[s], sem))
        copies.append(pltpu.async_copy(ib_hbm.at[iidx_v.at[s]], ibv.at[s], sem))
    for c in copies:
        c.wait()

    copies = []
    for j in range(NCH):
        s = pl.ds(j * CH, CH)
        copies.append(pltpu.async_copy(gu_hbm.at[wu_v.at[s]], ru_v.at[s], sem))
        copies.append(pltpu.async_copy(gi_hbm.at[wi_v.at[s]], ri_v.at[s], sem))
    for c in copies:
        c.wait()

    def fixup(idx_v, tail_hbm, rows_v):
        def vec_loop(i, carry):
            v = idx_v[pl.ds(i * L, L)]

            def cond(c):
                return jnp.any(c[0])

            def body(c):
                mm, n = c
                ffs = plsc.all_reduce_ffs(mm)
                rt = jnp.max(jnp.where(lane == ffs, v - TAILLO, -1))
                bl = i * L + jnp.max(ffs)
                pltpu.sync_copy(tail_hbm.at[rt], rows_v.at[bl])
                return mm & (lane != ffs), n

            lax.while_loop(cond, body, (v >= TAILLO, 0))
            return carry

        lax.fori_loop(0, BPW // L, vec_loop, 0)

    fixup(uidx_v, tu_hbm, ru_v)
    fixup(iidx_v, ti_hbm, ri_v)

    gb = gbv[...]

    def group(g, carry):
        rbase = g * L
        rows = rbase + lax.iota(jnp.int32, L)
        acc = ubv[pl.ds(rbase, L)] + ibv[pl.ds(rbase, L)] + gb
        dvec = jnp.zeros((L,), jnp.int32)
        for _ in range(D):
            du = plsc.load_gather(ru_v, [rows, dvec])
            di = plsc.load_gather(ri_v, [rows, dvec])
            acc = acc + du * di
            dvec = dvec + 1
        out_v[pl.ds(rbase, L)] = acc
        return carry

    lax.fori_loop(0, GROUPS, group, 0)

    pltpu.sync_copy(out_v, out_hbm.at[pl.ds(base, BPW)])


def kernel(user_indices, item_indices, user_embedding, item_embedding,
           user_bias, item_bias, global_bias):
    mesh = plsc.VectorSubcoreMesh(core_axis_name="c", subcore_axis_name="s")

    stream_k = pl.kernel(
        _stream_body,
        mesh=mesh,
        compiler_params=pltpu.CompilerParams(use_tc_tiling_on_sc=True,
                                             needs_layout_passes=False),
        out_type=(
            jax.ShapeDtypeStruct((RPAD,), jnp.int32),
            jax.ShapeDtypeStruct((RPAD,), jnp.int32),
            jax.ShapeDtypeStruct((BATCH * D,), jnp.float32),
            jax.ShapeDtypeStruct((BATCH * D,), jnp.float32),
        ),
        scratch_types=[
            pltpu.VMEM((LUTW,), jnp.int32),
            pltpu.VMEM((LUTW,), jnp.int32),
            pltpu.VMEM((IDXCH,), jnp.int32),
            [pltpu.VMEM((D, BLK), jnp.float32)] * 2,
            [pltpu.VMEM((D, BLK), jnp.float32)] * 2,
            pltpu.VMEM((RING * D,), jnp.float32),
            pltpu.VMEM((RING * D,), jnp.float32),
            pltpu.VMEM((NCNT,), jnp.int32),
            pltpu.VMEM((NCNT,), jnp.int32),
            pltpu.SemaphoreType.DMA,
            pltpu.SemaphoreType.DMA,
            pltpu.SemaphoreType.DMA,
        ],
    )

    dot_k = pl.kernel(
        _dot_body,
        mesh=mesh,
        compiler_params=pltpu.CompilerParams(use_tc_tiling_on_sc=False,
                                             needs_layout_passes=False),
        out_type=jax.ShapeDtypeStruct((BATCH,), jnp.float32),
        scratch_types=[
            pltpu.VMEM((BPW,), jnp.int32),
            pltpu.VMEM((BPW,), jnp.int32),
            pltpu.VMEM((BPW,), jnp.int32),
            pltpu.VMEM((BPW,), jnp.int32),
            pltpu.VMEM((BPW, D), jnp.float32),
            pltpu.VMEM((BPW, D), jnp.float32),
            pltpu.VMEM((BPW,), jnp.float32),
            pltpu.VMEM((BPW,), jnp.float32),
            pltpu.VMEM((L,), jnp.float32),
            pltpu.VMEM((BPW,), jnp.float32),
            pltpu.SemaphoreType.DMA,
        ],
    )

    uidx = user_indices.astype(jnp.int32)
    iidx = item_indices.astype(jnp.int32)
    win_u, win_i, gu, gi = stream_k(uidx, iidx,
                                    user_embedding.T, item_embedding.T)
    gu = gu.reshape(BATCH, D)
    gi = gi.reshape(BATCH, D)
    return dot_k(uidx, iidx, win_u, win_i, gu, gi,
                 user_embedding[TAILLO:, :], item_embedding[TAILLO:, :],
                 user_bias.reshape(-1), item_bias.reshape(-1),
                 jnp.broadcast_to(global_bias, (L,)))


# speedup vs baseline: 1.3850x; 1.3850x over previous
import jax
import jax.numpy as jnp
from jax import lax
from jax.experimental import pallas as pl
from jax.experimental.pallas import tpu as pltpu
from jax.experimental.pallas import tpu_sc as plsc

BATCH = 16384
D = 64
L = 16
NC, NS = 2, 16
NW = NC * NS
BPW = BATCH // NW
CH = 128
NCH = BPW // CH
GROUPS = BPW // L

NROW = 1_000_000
BLK = 128
NBLK = (NROW + BLK - 1) // BLK
RPAD = NBLK * BLK
MAXB = (NBLK + NW - 1) // NW
LUTW = MAXB * BLK
IDXCH = 2048
RING = 8
TAILLO = (NBLK - 1) * BLK
NCNT = 2048


def _harvest(lut, buf, stg, gout_hbm, sem_out, loff, k, mcount):
    slots = lut[pl.ds(loff + k * L, L)]
    lane = lax.iota(jnp.int32, L)

    def cond(c):
        return jnp.any(c[0])

    def body(c):
        m, n = c
        ffs = plsc.all_reduce_ffs(m)
        bvec = jnp.where(lane == ffs, slots, -1)
        b_out = jnp.max(bvec)
        lvec = k * L + ffs
        slot = lax.rem(n, RING)
        for c4 in range(4):
            dvec = c4 * L + lane
            vals = plsc.load_gather(buf, [dvec, lvec])
            stg[pl.ds(slot * D + c4 * L, L)] = vals

        @pl.when(n >= RING)
        def _():
            pltpu.make_async_copy(gout_hbm.at[pl.ds(0, D)],
                                  stg.at[pl.ds(0, D)], sem_out).wait()

        pltpu.async_copy(stg.at[pl.ds(slot * D, D)],
                         gout_hbm.at[pl.ds(b_out * D, D)], sem_out)
        return m & (lane != ffs), n + 1

    m0 = slots >= 0
    _, mcount = lax.while_loop(cond, body, (m0, mcount))
    return mcount


def _stream_body(uidx_hbm, iidx_hbm, utab_hbm, itab_hbm,
                 win_u_hbm, win_i_hbm, gu_hbm, gi_hbm,
                 lut_u, lut_i, chunk_v, bufs_u, bufs_i, stg_u, stg_i,
                 bf_u, bf_i,
                 sem_in, sem_u, sem_i):
    wid = lax.axis_index("s") * NC + lax.axis_index("c")
    blo = (wid * NBLK) // NW
    bhi = ((wid + 1) * NBLK) // NW
    nblk = bhi - blo
    rlo = blo * BLK
    lane = lax.iota(jnp.int32, L)

    neg1 = jnp.full((L,), -1, jnp.int32)
    zero = jnp.zeros((L,), jnp.int32)

    def init(i, carry):
        lut_u[pl.ds(i * L, L)] = neg1
        lut_i[pl.ds(i * L, L)] = neg1
        return carry

    lax.fori_loop(0, LUTW // L, init, 0)

    def initf(i, carry):
        bf_u[pl.ds(i * L, L)] = zero
        bf_i[pl.ds(i * L, L)] = zero
        return carry

    lax.fori_loop(0, NCNT // L, initf, 0)

    rhi = rlo + LUTW

    one = jnp.full((L,), 1, jnp.int32)

    def scan_tab(idx_hbm, lut, bf):
        def chunk_loop(c, carry):
            pltpu.sync_copy(idx_hbm.at[pl.ds(c * IDXCH, IDXCH)], chunk_v)

            def vec_loop(i, carry2):
                v = chunk_v[pl.ds(i * L, L)]
                m = (v >= rlo) & (v < rhi)
                bvec = c * IDXCH + i * L + lane
                plsc.store_scatter(lut, [v - rlo], bvec, mask=m)
                plsc.addupdate_scatter(
                    bf, [lax.shift_right_logical(v - rlo, 4)], one, mask=m)
                return carry2

            return lax.fori_loop(0, IDXCH // L, vec_loop, carry)

        lax.fori_loop(0, BATCH // IDXCH, chunk_loop, 0)

    scan_tab(uidx_hbm, lut_u, bf_u)
    scan_tab(iidx_hbm, lut_i, bf_i)

    def flag_at(bf, j):
        base = j * 8
        b16 = (base // L) * L
        cv = bf[pl.ds(b16, L)]
        off = base - b16
        return jnp.max(jnp.where((lane >= off) & (lane < off + 8), cv, 0))

    base = 31232
    pltpu.sync_copy(lut_u.at[pl.ds(0, base)], win_u_hbm.at[pl.ds(rlo, base)])
    pltpu.sync_copy(lut_i.at[pl.ds(0, base)], win_i_hbm.at[pl.ds(rlo, base)])

    @pl.when(nblk == MAXB)
    def _():
        pltpu.sync_copy(lut_u.at[pl.ds(base, BLK)],
                        win_u_hbm.at[pl.ds(rlo + base, BLK)])
        pltpu.sync_copy(lut_i.at[pl.ds(base, BLK)],
                        win_i_hbm.at[pl.ds(rlo + base, BLK)])

    bhi_s = jnp.minimum(bhi, NBLK - 1)

    def fire1(tab_hbm, g, pbuf):
        pltpu.async_copy(tab_hbm.at[:, pl.ds(g * BLK, BLK)], pbuf, sem_in)

    @pl.when(flag_at(bf_u, 0) > 0)
    def _():
        fire1(utab_hbm, blo, bufs_u[0])

    @pl.when(flag_at(bf_i, 0) > 0)
    def _():
        fire1(itab_hbm, blo, bufs_i[0])

    @pl.when((blo + 1 < bhi_s) & (flag_at(bf_u, 1) > 0))
    def _():
        fire1(utab_hbm, blo + 1, bufs_u[1])

    @pl.when((blo + 1 < bhi_s) & (flag_at(bf_i, 1) > 0))
    def _():
        fire1(itab_hbm, blo + 1, bufs_i[1])

    def blk_loop(g, carry):
        mu, mi = carry
        lb = g - blo
        fu = flag_at(bf_u, lb)
        fi = flag_at(bf_i, lb)

        @pl.when(fu > 0)
        def _():
            pltpu.make_async_copy(utab_hbm.at[pl.ds(0, 64), pl.ds(0, BLK)],
                                  bufs_u[0], sem_in).wait()

        @pl.when(fi > 0)
        def _():
            pltpu.make_async_copy(utab_hbm.at[pl.ds(0, 64), pl.ds(0, BLK)],
                                  bufs_i[0], sem_in).wait()

        p = lax.rem(lb, 3)

        def do(pbuf_u, pbuf_i, mu, mi):
            @pl.when((g + 2 < bhi_s) & (flag_at(bf_u, lb + 2) > 0))
            def _():
                fire1(utab_hbm, g + 2, pbuf_u)

            @pl.when((g + 2 < bhi_s) & (flag_at(bf_i, lb + 2) > 0))
            def _():
                fire1(itab_hbm, g + 2, pbuf_i)
            return mu, mi

        def proc(pbuf_u, pbuf_i, mu, mi):
            loff = lb * BLK

            def hloop(lut, pbuf, stg, gout, sem, bf):
                base = lb * 8
                b16 = (base // L) * L
                cv = bf[pl.ds(b16, L)]
                off = base - b16
                gm = (cv > 0) & (lane >= off) & (lane < off + 8)

                def cond(c):
                    return jnp.any(c[0])

                def body(c):
                    gmm, mc = c
                    gffs = plsc.all_reduce_ffs(gmm)
                    k = jnp.max(gffs) - off
                    mc = _harvest(lut, pbuf, stg, gout, sem, loff, k, mc)
                    return gmm & (lane != gffs), mc

                def run(c):
                    _, mc = lax.while_loop(cond, body, (gm, c))
                    return mc

                return run

            mu = lax.cond(fu > 0,
                          hloop(lut_u, pbuf_u, stg_u, gu_hbm, sem_u, bf_u),
                          lambda c: c, mu)
            mi = lax.cond(fi > 0,
                          hloop(lut_i, pbuf_i, stg_i, gi_hbm, sem_i, bf_i),
                          lambda c: c, mi)
            return mu, mi

        def br0(c):
            mu, mi = c
            mu, mi = do(bufs_u[2], bufs_i[2], mu, mi)
            return proc(bufs_u[0], bufs_i[0], mu, mi)

        def br1(c):
            mu, mi = c
            mu, mi = do(bufs_u[0], bufs_i[0], mu, mi)
            return proc(bufs_u[1], bufs_i[1], mu, mi)

        def br2(c):
            mu, mi = c
            mu, mi = do(bufs_u[1], bufs_i[1], mu, mi)
            return proc(bufs_u[2], bufs_i[2], mu, mi)

        return lax.cond(p == 0, br0,
                        lambda c: lax.cond(p == 1, br1, br2, c), (mu, mi))

    mu, mi = lax.fori_loop(blo, bhi_s, blk_loop,
                           (jnp.int32(0), jnp.int32(0)))

    def drain(sem, cnt, stg):
        def d(i, carry):
            pltpu.make_async_copy(gu_hbm.at[pl.ds(0, D)],
                                  stg.at[pl.ds(0, D)], sem).wait()
            return carry

        lax.fori_loop(0, jnp.minimum(cnt, RING), d, 0)

    drain(sem_u, mu, stg_u)
    drain(sem_i, mi, stg_i)


def _dot_body(uidx_hbm, iidx_hbm, win_u_hbm, win_i_hbm, gu_hbm, gi_hbm,
              tu_hbm, ti_hbm, ub_hbm, ib_hbm, gb_hbm, out_hbm,
              uidx_v, iidx_v, wu_v, wi_v, ru_v, ri_v, ubv, ibv, gbv, out_v,
              sem):
    wid = lax.axis_index("s") * NC + lax.axis_index("c")
    base = wid * BPW
    lane = lax.iota(jnp.int32, L)

    pltpu.sync_copy(uidx_hbm.at[pl.ds(base, BPW)], uidx_v)
    pltpu.sync_copy(iidx_hbm.at[pl.ds(base, BPW)], iidx_v)
    pltpu.sync_copy(gb_hbm, gbv)

    copies = []
    for j in range(NCH):
        s = pl.ds(j * CH, CH)
        copies.append(pltpu.async_copy(win_u_hbm.at[uidx_v.at[s]], wu_v.at[s], sem))
        copies.append(pltpu.async_copy(win_i_hbm.at[iidx_v.at[s]], wi_v.at[s], sem))
        copies.append(pltpu.async_copy(ub_hbm.at[uidx_v.at[s]], ubv.at[s], sem))
        copies.append(pltpu.async_copy(ib_hbm.at[iidx_v.at[s]], ibv.at[s], sem))
    for c in copies:
        c.wait()

    copies = []
    for j in range(NCH):
        s = pl.ds(j * CH, CH)
        copies.append(pltpu.async_copy(gu_hbm.at[wu_v.at[s]], ru_v.at[s], sem))
        copies.append(pltpu.async_copy(gi_hbm.at[wi_v.at[s]], ri_v.at[s], sem))
    for c in copies:
        c.wait()

    def fixup(idx_v, tail_hbm, rows_v):
        def vec_loop(i, carry):
            v = idx_v[pl.ds(i * L, L)]

            def cond(c):
                return jnp.any(c[0])

            def body(c):
                mm, n = c
                ffs = plsc.all_reduce_ffs(mm)
                rt = jnp.max(jnp.where(lane == ffs, v - TAILLO, -1))
                bl = i * L + jnp.max(ffs)
                pltpu.sync_copy(tail_hbm.at[rt], rows_v.at[bl])
                return mm & (lane != ffs), n

            lax.while_loop(cond, body, (v >= TAILLO, 0))
            return carry

        lax.fori_loop(0, BPW // L, vec_loop, 0)

    fixup(uidx_v, tu_hbm, ru_v)
    fixup(iidx_v, ti_hbm, ri_v)

    gb = gbv[...]

    def group(g, carry):
        rbase = g * L
        rows = rbase + lax.iota(jnp.int32, L)
        acc = ubv[pl.ds(rbase, L)] + ibv[pl.ds(rbase, L)] + gb
        dvec = jnp.zeros((L,), jnp.int32)
        for _ in range(D):
            du = plsc.load_gather(ru_v, [rows, dvec])
            di = plsc.load_gather(ri_v, [rows, dvec])
            acc = acc + du * di
            dvec = dvec + 1
        out_v[pl.ds(rbase, L)] = acc
        return carry

    lax.fori_loop(0, GROUPS, group, 0)

    pltpu.sync_copy(out_v, out_hbm.at[pl.ds(base, BPW)])


def kernel(user_indices, item_indices, user_embedding, item_embedding,
           user_bias, item_bias, global_bias):
    mesh = plsc.VectorSubcoreMesh(core_axis_name="c", subcore_axis_name="s")

    stream_k = pl.kernel(
        _stream_body,
        mesh=mesh,
        compiler_params=pltpu.CompilerParams(use_tc_tiling_on_sc=True,
                                             needs_layout_passes=False),
        out_type=(
            jax.ShapeDtypeStruct((RPAD,), jnp.int32),
            jax.ShapeDtypeStruct((RPAD,), jnp.int32),
            jax.ShapeDtypeStruct((BATCH * D,), jnp.float32),
            jax.ShapeDtypeStruct((BATCH * D,), jnp.float32),
        ),
        scratch_types=[
            pltpu.VMEM((LUTW,), jnp.int32),
            pltpu.VMEM((LUTW,), jnp.int32),
            pltpu.VMEM((IDXCH,), jnp.int32),
            [pltpu.VMEM((D, BLK), jnp.float32)] * 3,
            [pltpu.VMEM((D, BLK), jnp.float32)] * 3,
            pltpu.VMEM((RING * D,), jnp.float32),
            pltpu.VMEM((RING * D,), jnp.float32),
            pltpu.VMEM((NCNT,), jnp.int32),
            pltpu.VMEM((NCNT,), jnp.int32),
            pltpu.SemaphoreType.DMA,
            pltpu.SemaphoreType.DMA,
            pltpu.SemaphoreType.DMA,
        ],
    )

    dot_k = pl.kernel(
        _dot_body,
        mesh=mesh,
        compiler_params=pltpu.CompilerParams(use_tc_tiling_on_sc=False,
                                             needs_layout_passes=False),
        out_type=jax.ShapeDtypeStruct((BATCH,), jnp.float32),
        scratch_types=[
            pltpu.VMEM((BPW,), jnp.int32),
            pltpu.VMEM((BPW,), jnp.int32),
            pltpu.VMEM((BPW,), jnp.int32),
            pltpu.VMEM((BPW,), jnp.int32),
            pltpu.VMEM((BPW, D), jnp.float32),
            pltpu.VMEM((BPW, D), jnp.float32),
            pltpu.VMEM((BPW,), jnp.float32),
            pltpu.VMEM((BPW,), jnp.float32),
            pltpu.VMEM((L,), jnp.float32),
            pltpu.VMEM((BPW,), jnp.float32),
            pltpu.SemaphoreType.DMA,
        ],
    )

    uidx = user_indices.astype(jnp.int32)
    iidx = item_indices.astype(jnp.int32)
    win_u, win_i, gu, gi = stream_k(uidx, iidx,
                                    user_embedding.T, item_embedding.T)
    gu = gu.reshape(BATCH, D)
    gi = gi.reshape(BATCH, D)
    return dot_k(uidx, iidx, win_u, win_i, gu, gi,
                 user_embedding[TAILLO:, :], item_embedding[TAILLO:, :],
                 user_bias.reshape(-1), item_bias.reshape(-1),
                 jnp.broadcast_to(global_bias, (L,)))


# speedup vs baseline: 1.4079x; 1.0165x over previous
import jax
import jax.numpy as jnp
from jax import lax
from jax.experimental import pallas as pl
from jax.experimental.pallas import tpu as pltpu
from jax.experimental.pallas import tpu_sc as plsc

BATCH = 16384
D = 64
L = 16
NC, NS = 2, 16
NW = NC * NS
BPW = BATCH // NW
CH = 128
NCH = BPW // CH
GROUPS = BPW // L

NROW = 1_000_000
BLK = 128
NBLK = (NROW + BLK - 1) // BLK
RPAD = NBLK * BLK
MAXB = (NBLK + NW - 1) // NW
LUTW = MAXB * BLK
IDXCH = 2048
NBUF = 8
RING = 8
TAILLO = (NBLK - 1) * BLK
NCNT = 2048


def _harvest(lut, buf, stg, gout_hbm, sem_out, loff, k, mcount):
    slots = lut[pl.ds(loff + k * L, L)]
    lane = lax.iota(jnp.int32, L)

    def cond(c):
        return jnp.any(c[0])

    def body(c):
        m, n = c
        ffs = plsc.all_reduce_ffs(m)
        bvec = jnp.where(lane == ffs, slots, -1)
        b_out = jnp.max(bvec)
        lvec = k * L + ffs
        slot = lax.rem(n, RING)
        for c4 in range(4):
            dvec = c4 * L + lane
            vals = plsc.load_gather(buf, [dvec, lvec])
            stg[pl.ds(slot * D + c4 * L, L)] = vals

        @pl.when(n >= RING)
        def _():
            pltpu.make_async_copy(gout_hbm.at[pl.ds(0, D)],
                                  stg.at[pl.ds(0, D)], sem_out).wait()

        pltpu.async_copy(stg.at[pl.ds(slot * D, D)],
                         gout_hbm.at[pl.ds(b_out * D, D)], sem_out)
        return m & (lane != ffs), n + 1

    m0 = slots >= 0
    _, mcount = lax.while_loop(cond, body, (m0, mcount))
    return mcount


def _stream_body(uidx_hbm, iidx_hbm, utab_hbm, itab_hbm,
                 win_u_hbm, win_i_hbm, gu_hbm, gi_hbm,
                 lut, chunk_v, bufs, stg, bf,
                 sem_in, sem_out):
    wid = lax.axis_index("s") * NC + lax.axis_index("c")
    blo = (wid * NBLK) // NW
    bhi = ((wid + 1) * NBLK) // NW
    nblk = bhi - blo
    rlo = blo * BLK
    rhi = rlo + LUTW
    lane = lax.iota(jnp.int32, L)
    bhi_s = jnp.minimum(bhi, NBLK - 1)

    neg1 = jnp.full((L,), -1, jnp.int32)
    zero = jnp.zeros((L,), jnp.int32)
    one = jnp.full((L,), 1, jnp.int32)

    def flag_at(j):
        base = j * 8
        b16 = (base // L) * L
        cv = bf[pl.ds(b16, L)]
        off = base - b16
        return jnp.max(jnp.where((lane >= off) & (lane < off + 8), cv, 0))

    def fire1(tab_hbm, g, pbuf):
        pltpu.async_copy(tab_hbm.at[:, pl.ds(g * BLK, BLK)], pbuf, sem_in)

    def phase(idx_hbm, tab_hbm, win_hbm, gout_hbm):
        def init(i, carry):
            lut[pl.ds(i * L, L)] = neg1
            return carry

        lax.fori_loop(0, LUTW // L, init, 0)

        def initf(i, carry):
            bf[pl.ds(i * L, L)] = zero
            return carry

        lax.fori_loop(0, NCNT // L, initf, 0)

        def chunk_loop(c, carry):
            pltpu.sync_copy(idx_hbm.at[pl.ds(c * IDXCH, IDXCH)], chunk_v)

            def vec_loop(i, carry2):
                v = chunk_v[pl.ds(i * L, L)]
                m = (v >= rlo) & (v < rhi)
                bvec = c * IDXCH + i * L + lane
                plsc.store_scatter(lut, [v - rlo], bvec, mask=m)
                plsc.addupdate_scatter(
                    bf, [lax.shift_right_logical(v - rlo, 4)], one, mask=m)
                return carry2

            return lax.fori_loop(0, IDXCH // L, vec_loop, carry)

        lax.fori_loop(0, BATCH // IDXCH, chunk_loop, 0)

        base = (MAXB - 1) * BLK
        pltpu.sync_copy(lut.at[pl.ds(0, base)], win_hbm.at[pl.ds(rlo, base)])

        @pl.when(nblk == MAXB)
        def _():
            pltpu.sync_copy(lut.at[pl.ds(base, BLK)],
                            win_hbm.at[pl.ds(rlo + base, BLK)])

        for j in range(NBUF):
            @pl.when((blo + j < bhi_s) & (flag_at(j) > 0))
            def _(j=j):
                fire1(tab_hbm, blo + j, bufs[j])

        def hloop(pbuf, lb):
            base = lb * 8
            b16 = (base // L) * L
            cv = bf[pl.ds(b16, L)]
            off = base - b16
            gm = (cv > 0) & (lane >= off) & (lane < off + 8)
            loff = lb * BLK

            def cond(c):
                return jnp.any(c[0])

            def body(c):
                gmm, mc = c
                gffs = plsc.all_reduce_ffs(gmm)
                k = jnp.max(gffs) - off
                mc = _harvest(lut, pbuf, stg, gout_hbm, sem_out, loff, k, mc)
                return gmm & (lane != gffs), mc

            def run(c):
                _, mc = lax.while_loop(cond, body, (gm, c))
                return mc

            return run

        def blk_loop(g, mc):
            lb = g - blo
            f = flag_at(lb)
            p = lax.rem(lb, NBUF)

            def slot(j):
                def br(c):
                    @pl.when(f > 0)
                    def _():
                        pltpu.make_async_copy(
                            tab_hbm.at[pl.ds(0, 64), pl.ds(0, BLK)],
                            bufs[j], sem_in).wait()

                    c2 = lax.cond(f > 0, hloop(bufs[j], lb), lambda cc: cc, c)

                    @pl.when((g + NBUF < bhi_s) & (flag_at(lb + NBUF) > 0))
                    def _():
                        fire1(tab_hbm, g + NBUF, bufs[j])

                    return c2

                return br

            def dispatch(lo, hi, c):
                if hi - lo == 1:
                    return slot(lo)(c)
                mid = (lo + hi) // 2
                return lax.cond(p < mid,
                                lambda cc: dispatch(lo, mid, cc),
                                lambda cc: dispatch(mid, hi, cc), c)

            return dispatch(0, NBUF, mc)

        mc = lax.fori_loop(blo, bhi_s, blk_loop, jnp.int32(0))

        def d(i, carry):
            pltpu.make_async_copy(gout_hbm.at[pl.ds(0, D)],
                                  stg.at[pl.ds(0, D)], sem_out).wait()
            return carry

        lax.fori_loop(0, jnp.minimum(mc, RING), d, 0)

    phase(uidx_hbm, utab_hbm, win_u_hbm, gu_hbm)
    phase(iidx_hbm, itab_hbm, win_i_hbm, gi_hbm)


def _dot_body(uidx_hbm, iidx_hbm, win_u_hbm, win_i_hbm, gu_hbm, gi_hbm,
              tu_hbm, ti_hbm, ub_hbm, ib_hbm, gb_hbm, out_hbm,
              uidx_v, iidx_v, wu_v, wi_v, ru_v, ri_v, ubv, ibv, gbv, out_v,
              sem):
    wid = lax.axis_index("s") * NC + lax.axis_index("c")
    base = wid * BPW
    lane = lax.iota(jnp.int32, L)

    pltpu.sync_copy(uidx_hbm.at[pl.ds(base, BPW)], uidx_v)
    pltpu.sync_copy(iidx_hbm.at[pl.ds(base, BPW)], iidx_v)
    pltpu.sync_copy(gb_hbm, gbv)

    copies = []
    for j in range(NCH):
        s = pl.ds(j * CH, CH)
        copies.append(pltpu.async_copy(win_u_hbm.at[uidx_v.at[s]], wu_v.at[s], sem))
        copies.append(pltpu.async_copy(win_i_hbm.at[iidx_v.at[s]], wi_v.at[s], sem))
        copies.append(pltpu.async_copy(ub_hbm.at[uidx_v.at[s]], ubv.at[s], sem))
        copies.append(pltpu.async_copy(ib_hbm.at[iidx_v.at[s]], ibv.at[s], sem))
    for c in copies:
        c.wait()

    copies = []
    for j in range(NCH):
        s = pl.ds(j * CH, CH)
        copies.append(pltpu.async_copy(gu_hbm.at[wu_v.at[s]], ru_v.at[s], sem))
        copies.append(pltpu.async_copy(gi_hbm.at[wi_v.at[s]], ri_v.at[s], sem))
    for c in copies:
        c.wait()

    def fixup(idx_v, tail_hbm, rows_v):
        def vec_loop(i, carry):
            v = idx_v[pl.ds(i * L, L)]

            def cond(c):
                return jnp.any(c[0])

            def body(c):
                mm, n = c
                ffs = plsc.all_reduce_ffs(mm)
                rt = jnp.max(jnp.where(lane == ffs, v - TAILLO, -1))
                bl = i * L + jnp.max(ffs)
                pltpu.sync_copy(tail_hbm.at[rt], rows_v.at[bl])
                return mm & (lane != ffs), n

            lax.while_loop(cond, body, (v >= TAILLO, 0))
            return carry

        lax.fori_loop(0, BPW // L, vec_loop, 0)

    fixup(uidx_v, tu_hbm, ru_v)
    fixup(iidx_v, ti_hbm, ri_v)

    gb = gbv[...]

    def group(g, carry):
        rbase = g * L
        rows = rbase + lax.iota(jnp.int32, L)
        acc = ubv[pl.ds(rbase, L)] + ibv[pl.ds(rbase, L)] + gb
        dvec = jnp.zeros((L,), jnp.int32)
        for _ in range(D):
            du = plsc.load_gather(ru_v, [rows, dvec])
            di = plsc.load_gather(ri_v, [rows, dvec])
            acc = acc + du * di
            dvec = dvec + 1
        out_v[pl.ds(rbase, L)] = acc
        return carry

    lax.fori_loop(0, GROUPS, group, 0)

    pltpu.sync_copy(out_v, out_hbm.at[pl.ds(base, BPW)])


def kernel(user_indices, item_indices, user_embedding, item_embedding,
           user_bias, item_bias, global_bias):
    mesh = plsc.VectorSubcoreMesh(core_axis_name="c", subcore_axis_name="s")

    stream_k = pl.kernel(
        _stream_body,
        mesh=mesh,
        compiler_params=pltpu.CompilerParams(use_tc_tiling_on_sc=True,
                                             needs_layout_passes=False),
        out_type=(
            jax.ShapeDtypeStruct((RPAD,), jnp.int32),
            jax.ShapeDtypeStruct((RPAD,), jnp.int32),
            jax.ShapeDtypeStruct((BATCH * D,), jnp.float32),
            jax.ShapeDtypeStruct((BATCH * D,), jnp.float32),
        ),
        scratch_types=[
            pltpu.VMEM((LUTW,), jnp.int32),
            pltpu.VMEM((IDXCH,), jnp.int32),
            [pltpu.VMEM((D, BLK), jnp.float32)] * NBUF,
            pltpu.VMEM((RING * D,), jnp.float32),
            pltpu.VMEM((NCNT,), jnp.int32),
            pltpu.SemaphoreType.DMA,
            pltpu.SemaphoreType.DMA,
        ],
    )

    dot_k = pl.kernel(
        _dot_body,
        mesh=mesh,
        compiler_params=pltpu.CompilerParams(use_tc_tiling_on_sc=False,
                                             needs_layout_passes=False),
        out_type=jax.ShapeDtypeStruct((BATCH,), jnp.float32),
        scratch_types=[
            pltpu.VMEM((BPW,), jnp.int32),
            pltpu.VMEM((BPW,), jnp.int32),
            pltpu.VMEM((BPW,), jnp.int32),
            pltpu.VMEM((BPW,), jnp.int32),
            pltpu.VMEM((BPW, D), jnp.float32),
            pltpu.VMEM((BPW, D), jnp.float32),
            pltpu.VMEM((BPW,), jnp.float32),
            pltpu.VMEM((BPW,), jnp.float32),
            pltpu.VMEM((L,), jnp.float32),
            pltpu.VMEM((BPW,), jnp.float32),
            pltpu.SemaphoreType.DMA,
        ],
    )

    uidx = user_indices.astype(jnp.int32)
    iidx = item_indices.astype(jnp.int32)
    win_u, win_i, gu, gi = stream_k(uidx, iidx,
                                    user_embedding.T, item_embedding.T)
    gu = gu.reshape(BATCH, D)
    gi = gi.reshape(BATCH, D)
    return dot_k(uidx, iidx, win_u, win_i, gu, gi,
                 user_embedding[TAILLO:, :], item_embedding[TAILLO:, :],
                 user_bias.reshape(-1), item_bias.reshape(-1),
                 jnp.broadcast_to(global_bias, (L,)))


# speedup vs baseline: 1.4778x; 1.0497x over previous
import jax
import jax.numpy as jnp
from jax import lax
from jax.experimental import pallas as pl
from jax.experimental.pallas import tpu as pltpu
from jax.experimental.pallas import tpu_sc as plsc

BATCH = 16384
D = 64
L = 16
NC, NS = 2, 16
NW = NC * NS
BPW = BATCH // NW
CH = 128
NCH = BPW // CH
GROUPS = BPW // L

NROW = 1_000_000
BLK = 128
NBLK = (NROW + BLK - 1) // BLK
RPAD = NBLK * BLK
MAXB = (NBLK + NW - 1) // NW
LUTW = MAXB * BLK
IDXCH = 2048
NBUF = 8
HLF = 64
STG = 2 * HLF
CAPW = BATCH + HLF
TAILLO = (NBLK - 1) * BLK
NCNT = 2048


def _harvest(lut, buf, stg, gout_hbm, sem_out, wbase, loff, k, mcount):
    slots = lut[pl.ds(loff + k * L, L)]
    lane = lax.iota(jnp.int32, L)

    def cond(c):
        return jnp.any(c[0])

    def body(c):
        m, n = c
        ffs = plsc.all_reduce_ffs(m)
        s = lax.rem(n, STG)

        @pl.when((lax.rem(n, HLF) == 0) & (n >= STG))
        def _():
            pltpu.make_async_copy(gout_hbm.at[pl.ds(0, HLF * D)],
                                  stg.at[pl.ds(0, HLF * D)], sem_out).wait()

        lvec = k * L + ffs
        for c4 in range(4):
            dvec = c4 * L + lane
            vals = plsc.load_gather(buf, [dvec, lvec])
            stg[pl.ds(s * D + c4 * L, L)] = vals

        posv = jnp.full((L,), wbase, jnp.int32) + n
        plsc.store_scatter(lut, [loff + k * L + ffs], posv, mask=(lane == ffs))

        n1 = n + 1

        @pl.when(lax.rem(n1, HLF) == 0)
        def _():
            hrow = lax.rem(n1 - HLF, STG)
            pltpu.async_copy(
                stg.at[pl.ds(hrow * D, HLF * D)],
                gout_hbm.at[pl.ds((wbase + n1 - HLF) * D, HLF * D)], sem_out)

        return m & (lane != ffs), n1

    m0 = slots >= 0
    _, mcount = lax.while_loop(cond, body, (m0, mcount))
    return mcount


def _stream_body(uidx_hbm, iidx_hbm, utab_hbm, itab_hbm,
                 win_u_hbm, win_i_hbm, gu_hbm, gi_hbm,
                 lut, chunk_v, bufs, stg, bf,
                 sem_in, sem_out):
    wid = lax.axis_index("s") * NC + lax.axis_index("c")
    blo = (wid * NBLK) // NW
    bhi = ((wid + 1) * NBLK) // NW
    nblk = bhi - blo
    rlo = blo * BLK
    rhi = rlo + LUTW
    lane = lax.iota(jnp.int32, L)
    bhi_s = jnp.minimum(bhi, NBLK - 1)

    neg1 = jnp.full((L,), -1, jnp.int32)
    zero = jnp.zeros((L,), jnp.int32)
    one = jnp.full((L,), 1, jnp.int32)

    def flag_at(j):
        base = j * 8
        b16 = (base // L) * L
        cv = bf[pl.ds(b16, L)]
        off = base - b16
        return jnp.max(jnp.where((lane >= off) & (lane < off + 8), cv, 0))

    def fire1(tab_hbm, g, pbuf):
        pltpu.async_copy(tab_hbm.at[:, pl.ds(g * BLK, BLK)], pbuf, sem_in)

    def phase(idx_hbm, tab_hbm, win_hbm, gout_hbm):
        def init(i, carry):
            lut[pl.ds(i * L, L)] = neg1
            return carry

        lax.fori_loop(0, LUTW // L, init, 0)

        def initf(i, carry):
            bf[pl.ds(i * L, L)] = zero
            return carry

        lax.fori_loop(0, NCNT // L, initf, 0)

        def chunk_loop(c, carry):
            pltpu.sync_copy(idx_hbm.at[pl.ds(c * IDXCH, IDXCH)], chunk_v)

            def vec_loop(i, carry2):
                v = chunk_v[pl.ds(i * L, L)]
                m = (v >= rlo) & (v < rhi)
                bvec = c * IDXCH + i * L + lane
                plsc.store_scatter(lut, [v - rlo], bvec, mask=m)
                plsc.addupdate_scatter(
                    bf, [lax.shift_right_logical(v - rlo, 4)], one, mask=m)
                return carry2

            return lax.fori_loop(0, IDXCH // L, vec_loop, carry)

        lax.fori_loop(0, BATCH // IDXCH, chunk_loop, 0)

        wbase = wid * CAPW

        for j in range(NBUF):
            @pl.when((blo + j < bhi_s) & (flag_at(j) > 0))
            def _(j=j):
                fire1(tab_hbm, blo + j, bufs[j])

        def hloop(pbuf, lb):
            base = lb * 8
            b16 = (base // L) * L
            cv = bf[pl.ds(b16, L)]
            off = base - b16
            gm = (cv > 0) & (lane >= off) & (lane < off + 8)
            loff = lb * BLK

            def cond(c):
                return jnp.any(c[0])

            def body(c):
                gmm, mc = c
                gffs = plsc.all_reduce_ffs(gmm)
                k = jnp.max(gffs) - off
                mc = _harvest(lut, pbuf, stg, gout_hbm, sem_out,
                              wbase, loff, k, mc)
                return gmm & (lane != gffs), mc

            def run(c):
                _, mc = lax.while_loop(cond, body, (gm, c))
                return mc

            return run

        def blk_loop(g, mc):
            lb = g - blo
            f = flag_at(lb)
            p = lax.rem(lb, NBUF)

            def slot(j):
                def br(c):
                    @pl.when(f > 0)
                    def _():
                        pltpu.make_async_copy(
                            tab_hbm.at[pl.ds(0, 64), pl.ds(0, BLK)],
                            bufs[j], sem_in).wait()

                    c2 = lax.cond(f > 0, hloop(bufs[j], lb), lambda cc: cc, c)

                    @pl.when((g + NBUF < bhi_s) & (flag_at(lb + NBUF) > 0))
                    def _():
                        fire1(tab_hbm, g + NBUF, bufs[j])

                    return c2

                return br

            def dispatch(lo, hi, c):
                if hi - lo == 1:
                    return slot(lo)(c)
                mid = (lo + hi) // 2
                return lax.cond(p < mid,
                                lambda cc: dispatch(lo, mid, cc),
                                lambda cc: dispatch(mid, hi, cc), c)

            return dispatch(0, NBUF, mc)

        mc = lax.fori_loop(blo, bhi_s, blk_loop, jnp.int32(0))

        nout = mc // HLF - jnp.maximum(0, (mc - 1) // HLF - 1)

        def dwait(i, carry):
            pltpu.make_async_copy(gout_hbm.at[pl.ds(0, HLF * D)],
                                  stg.at[pl.ds(0, HLF * D)], sem_out).wait()
            return carry

        lax.fori_loop(0, nout, dwait, 0)

        base_n = mc - lax.rem(mc, HLF)
        hrow = lax.rem(base_n, STG)
        pltpu.sync_copy(stg.at[pl.ds(hrow * D, HLF * D)],
                        gout_hbm.at[pl.ds((wbase + base_n) * D, HLF * D)])

        pbase = (MAXB - 1) * BLK
        pltpu.sync_copy(lut.at[pl.ds(0, pbase)], win_hbm.at[pl.ds(rlo, pbase)])

        @pl.when(nblk == MAXB)
        def _():
            pltpu.sync_copy(lut.at[pl.ds(pbase, BLK)],
                            win_hbm.at[pl.ds(rlo + pbase, BLK)])

    phase(uidx_hbm, utab_hbm, win_u_hbm, gu_hbm)
    phase(iidx_hbm, itab_hbm, win_i_hbm, gi_hbm)


def _dot_body(uidx_hbm, iidx_hbm, win_u_hbm, win_i_hbm, gu_hbm, gi_hbm,
              tu_hbm, ti_hbm, ub_hbm, ib_hbm, gb_hbm, out_hbm,
              uidx_v, iidx_v, wu_v, wi_v, ru_v, ri_v, ubv, ibv, gbv, out_v,
              sem):
    wid = lax.axis_index("s") * NC + lax.axis_index("c")
    base = wid * BPW
    lane = lax.iota(jnp.int32, L)

    pltpu.sync_copy(uidx_hbm.at[pl.ds(base, BPW)], uidx_v)
    pltpu.sync_copy(iidx_hbm.at[pl.ds(base, BPW)], iidx_v)
    pltpu.sync_copy(gb_hbm, gbv)

    copies = []
    for j in range(NCH):
        s = pl.ds(j * CH, CH)
        copies.append(pltpu.async_copy(win_u_hbm.at[uidx_v.at[s]], wu_v.at[s], sem))
        copies.append(pltpu.async_copy(win_i_hbm.at[iidx_v.at[s]], wi_v.at[s], sem))
        copies.append(pltpu.async_copy(ub_hbm.at[uidx_v.at[s]], ubv.at[s], sem))
        copies.append(pltpu.async_copy(ib_hbm.at[iidx_v.at[s]], ibv.at[s], sem))
    for c in copies:
        c.wait()

    copies = []
    for j in range(NCH):
        s = pl.ds(j * CH, CH)
        copies.append(pltpu.async_copy(gu_hbm.at[wu_v.at[s]], ru_v.at[s], sem))
        copies.append(pltpu.async_copy(gi_hbm.at[wi_v.at[s]], ri_v.at[s], sem))
    for c in copies:
        c.wait()

    def fixup(idx_v, tail_hbm, rows_v):
        def vec_loop(i, carry):
            v = idx_v[pl.ds(i * L, L)]

            def cond(c):
                return jnp.any(c[0])

            def body(c):
                mm, n = c
                ffs = plsc.all_reduce_ffs(mm)
                rt = jnp.max(jnp.where(lane == ffs, v - TAILLO, -1))
                bl = i * L + jnp.max(ffs)
                pltpu.sync_copy(tail_hbm.at[rt], rows_v.at[bl])
                return mm & (lane != ffs), n

            lax.while_loop(cond, body, (v >= TAILLO, 0))
            return carry

        lax.fori_loop(0, BPW // L, vec_loop, 0)

    fixup(uidx_v, tu_hbm, ru_v)
    fixup(iidx_v, ti_hbm, ri_v)

    gb = gbv[...]

    def group(g, carry):
        rbase = g * L
        rows = rbase + lax.iota(jnp.int32, L)
        acc = ubv[pl.ds(rbase, L)] + ibv[pl.ds(rbase, L)] + gb
        dvec = jnp.zeros((L,), jnp.int32)
        for _ in range(D):
            du = plsc.load_gather(ru_v, [rows, dvec])
            di = plsc.load_gather(ri_v, [rows, dvec])
            acc = acc + du * di
            dvec = dvec + 1
        out_v[pl.ds(rbase, L)] = acc
        return carry

    lax.fori_loop(0, GROUPS, group, 0)

    pltpu.sync_copy(out_v, out_hbm.at[pl.ds(base, BPW)])


def kernel(user_indices, item_indices, user_embedding, item_embedding,
           user_bias, item_bias, global_bias):
    mesh = plsc.VectorSubcoreMesh(core_axis_name="c", subcore_axis_name="s")

    stream_k = pl.kernel(
        _stream_body,
        mesh=mesh,
        compiler_params=pltpu.CompilerParams(use_tc_tiling_on_sc=True,
                                             needs_layout_passes=False),
        out_type=(
            jax.ShapeDtypeStruct((RPAD,), jnp.int32),
            jax.ShapeDtypeStruct((RPAD,), jnp.int32),
            jax.ShapeDtypeStruct((NW * CAPW * D,), jnp.float32),
            jax.ShapeDtypeStruct((NW * CAPW * D,), jnp.float32),
        ),
        scratch_types=[
            pltpu.VMEM((LUTW,), jnp.int32),
            pltpu.VMEM((IDXCH,), jnp.int32),
            [pltpu.VMEM((D, BLK), jnp.float32)] * NBUF,
            pltpu.VMEM((STG * D,), jnp.float32),
            pltpu.VMEM((NCNT,), jnp.int32),
            pltpu.SemaphoreType.DMA,
            pltpu.SemaphoreType.DMA,
        ],
    )

    dot_k = pl.kernel(
        _dot_body,
        mesh=mesh,
        compiler_params=pltpu.CompilerParams(use_tc_tiling_on_sc=False,
                                             needs_layout_passes=False),
        out_type=jax.ShapeDtypeStruct((BATCH,), jnp.float32),
        scratch_types=[
            pltpu.VMEM((BPW,), jnp.int32),
            pltpu.VMEM((BPW,), jnp.int32),
            pltpu.VMEM((BPW,), jnp.int32),
            pltpu.VMEM((BPW,), jnp.int32),
            pltpu.VMEM((BPW, D), jnp.float32),
            pltpu.VMEM((BPW, D), jnp.float32),
            pltpu.VMEM((BPW,), jnp.float32),
            pltpu.VMEM((BPW,), jnp.float32),
            pltpu.VMEM((L,), jnp.float32),
            pltpu.VMEM((BPW,), jnp.float32),
            pltpu.SemaphoreType.DMA,
        ],
    )

    uidx = user_indices.astype(jnp.int32)
    iidx = item_indices.astype(jnp.int32)
    win_u, win_i, gu, gi = stream_k(uidx, iidx,
                                    user_embedding.T, item_embedding.T)
    gu = gu.reshape(NW * CAPW, D)
    gi = gi.reshape(NW * CAPW, D)
    return dot_k(uidx, iidx, win_u, win_i, gu, gi,
                 user_embedding[TAILLO:, :], item_embedding[TAILLO:, :],
                 user_bias.reshape(-1), item_bias.reshape(-1),
                 jnp.broadcast_to(global_bias, (L,)))
